# baseline (speedup 1.0000x reference)
import functools

import jax
import jax.numpy as jnp
from jax import lax
from jax.experimental import pallas as pl
from jax.experimental.pallas import tpu as pltpu
from jax.experimental.pallas import tpu_sc as plsc

B, A, F = 256, 128, 128
R = 16
NCHUNK = A // R
NW = 32
GPW = B // NW
NV = F // 16
ZR = 64
NBUF = 4


def _sc_body(x_hbm, ms_hbm, prev_hbm, z_hbm, out_hbm,
             ms_v, xb, p0b, p1b, ob, ms_s, sem_in, sem_out, sem_z):
    wid = lax.axis_index("s") * 2 + lax.axis_index("c")
    g0 = pl.multiple_of(wid * GPW, GPW)
    pltpu.sync_copy(ms_hbm.at[pl.ds(g0 * 2, 2 * GPW)], ms_v)
    mvec = ms_v[...]
    for i in range(GPW):
        ms_s[i] = mvec[2 * i]

    def graph_body(i, _):
        g = g0 + i
        m = ms_s[i]
        nvc = (m + R - 1) // R
        t = A - nvc * R
        base = pl.multiple_of(nvc * R, R)
        off32 = pl.multiple_of(base + (t & 64), R)
        off16 = pl.multiple_of(off32 + (t & 32), R)

        def ztail(op):
            def z64():
                op(pltpu.make_async_copy(
                    z_hbm.at[pl.ds(0, 64), :],
                    out_hbm.at[g, pl.ds(base, 64), :], sem_z))
            pl.when((t & 64) != 0)(z64)

            def z32():
                op(pltpu.make_async_copy(
                    z_hbm.at[pl.ds(0, 32), :],
                    out_hbm.at[g, pl.ds(off32, 32), :], sem_z))
            pl.when((t & 32) != 0)(z32)

            def z16():
                op(pltpu.make_async_copy(
                    z_hbm.at[pl.ds(0, 16), :],
                    out_hbm.at[g, pl.ds(off16, 16), :], sem_z))
            pl.when((t & 16) != 0)(z16)

        ztail(lambda cp: cp.start())

        def in_copies(c, b):
            r0 = pl.multiple_of(c * R, R)
            return (
                pltpu.make_async_copy(x_hbm.at[g, pl.ds(r0, R), :],
                                      xb.at[b], sem_in.at[b]),
                pltpu.make_async_copy(prev_hbm.at[0, g, pl.ds(r0, R), :],
                                      p0b.at[b], sem_in.at[b]),
                pltpu.make_async_copy(prev_hbm.at[1, g, pl.ds(r0, R), :],
                                      p1b.at[b], sem_in.at[b]),
            )

        def out_copy(c, b):
            return pltpu.make_async_copy(
                ob.at[b], out_hbm.at[g, pl.ds(pl.multiple_of(c * R, R), R), :],
                sem_out.at[b])

        for cpre in range(NBUF - 1):
            def prime(cpre=cpre):
                for cp in in_copies(cpre, cpre):
                    cp.start()
            pl.when(cpre < nvc)(prime)

        def chunk_body(c, _):
            b = lax.rem(c, NBUF)

            def prefetch():
                for cp in in_copies(c + NBUF - 1, lax.rem(c + NBUF - 1, NBUF)):
                    cp.start()
            pl.when(c + NBUF - 1 < nvc)(prefetch)

            for cp in in_copies(c, b):
                cp.wait()

            def drain_prev_out():
                out_copy(c - NBUF, b).wait()
            pl.when(c >= NBUF)(drain_prev_out)

            r0 = c * R

            @plsc.parallel_loop(0, R, step=1, unroll=4)
            def row_body(j):
                valid = (r0 + j) < m
                for k in range(NV):
                    sl = pl.ds(k * 16, 16)
                    v = xb[b, j, sl] + p0b[b, j, sl] + p1b[b, j, sl]
                    ob[b, j, sl] = jnp.where(valid, jnp.maximum(v, 0.0), 0.0)

            out_copy(c, b).start()
            return 0

        lax.fori_loop(0, nvc, chunk_body, 0)

        for k in range(NBUF, 0, -1):
            def drain(k=k):
                out_copy(nvc - k, lax.rem(nvc - k, NBUF)).wait()
            pl.when(nvc >= k)(drain)

        ztail(lambda cp: cp.wait())
        return 0

    lax.fori_loop(0, GPW, graph_body, 0)


def kernel(atom_features, mol_slice, prev_activations):
    mesh = plsc.VectorSubcoreMesh(core_axis_name="c", subcore_axis_name="s")
    run = functools.partial(
        pl.kernel,
        mesh=mesh,
        out_type=jax.ShapeDtypeStruct((B, A, F), jnp.float32),
        scratch_types=[
            pltpu.VMEM((2 * GPW,), jnp.int32),
            pltpu.VMEM((NBUF, R, F), jnp.float32),
            pltpu.VMEM((NBUF, R, F), jnp.float32),
            pltpu.VMEM((NBUF, R, F), jnp.float32),
            pltpu.VMEM((NBUF, R, F), jnp.float32),
            pltpu.SMEM((GPW,), jnp.int32),
            pltpu.SemaphoreType.DMA((NBUF,)),
            pltpu.SemaphoreType.DMA((NBUF,)),
            pltpu.SemaphoreType.DMA,
        ],
    )(_sc_body)
    zeros = jnp.zeros((ZR, F), jnp.float32)
    return run(atom_features, mol_slice.reshape(-1), prev_activations, zeros)


# speedup vs baseline: 4.6668x; 4.6668x over previous
import functools

import jax
import jax.numpy as jnp
from jax import lax
from jax.experimental import pallas as pl
from jax.experimental.pallas import tpu as pltpu
from jax.experimental.pallas import tpu_sc as plsc

B, A, F = 256, 128, 128
R = 16
NCHUNK = A // R
NW = 32
GPW = B // NW
NV = F // 16
ZR = 64
NBUF = 4


def _sc_body(x_hbm, ms_hbm, prev_hbm, out_hbm,
             ms_v, xb, p0b, p1b, ob, zb, ms_s, sem_in, sem_out, sem_z):
    wid = lax.axis_index("s") * 2 + lax.axis_index("c")
    g0 = pl.multiple_of(wid * GPW, GPW)
    pltpu.sync_copy(ms_hbm.at[pl.ds(g0 * 2, 2 * GPW)], ms_v)
    mvec = ms_v[...]
    for i in range(GPW):
        ms_s[i] = mvec[2 * i]

    zvec = jnp.zeros((16,), jnp.float32)
    for j in range(ZR):
        for k in range(NV):
            zb[j, pl.ds(k * 16, 16)] = zvec

    def graph_body(i, _):
        g = g0 + i
        m = ms_s[i]
        nvc = (m + R - 1) // R
        t = A - nvc * R
        base = pl.multiple_of(nvc * R, R)
        off32 = pl.multiple_of(base + (t & 64), R)
        off16 = pl.multiple_of(off32 + (t & 32), R)

        def ztail(op):
            def z64():
                op(pltpu.make_async_copy(
                    zb.at[pl.ds(0, 64), :],
                    out_hbm.at[g, pl.ds(base, 64), :], sem_z))
            pl.when((t & 64) != 0)(z64)

            def z32():
                op(pltpu.make_async_copy(
                    zb.at[pl.ds(0, 32), :],
                    out_hbm.at[g, pl.ds(off32, 32), :], sem_z))
            pl.when((t & 32) != 0)(z32)

            def z16():
                op(pltpu.make_async_copy(
                    zb.at[pl.ds(0, 16), :],
                    out_hbm.at[g, pl.ds(off16, 16), :], sem_z))
            pl.when((t & 16) != 0)(z16)

        ztail(lambda cp: cp.start())

        def in_copies(c, b):
            r0 = pl.multiple_of(c * R, R)
            return (
                pltpu.make_async_copy(x_hbm.at[g, pl.ds(r0, R), :],
                                      xb.at[b], sem_in.at[b]),
                pltpu.make_async_copy(prev_hbm.at[0, g, pl.ds(r0, R), :],
                                      p0b.at[b], sem_in.at[b]),
                pltpu.make_async_copy(prev_hbm.at[1, g, pl.ds(r0, R), :],
                                      p1b.at[b], sem_in.at[b]),
            )

        def out_copy(c, b):
            return pltpu.make_async_copy(
                ob.at[b], out_hbm.at[g, pl.ds(pl.multiple_of(c * R, R), R), :],
                sem_out.at[b])

        for cpre in range(NBUF - 1):
            def prime(cpre=cpre):
                for cp in in_copies(cpre, cpre):
                    cp.start()
            pl.when(cpre < nvc)(prime)

        def chunk_body(c, _):
            b = lax.rem(c, NBUF)

            def prefetch():
                for cp in in_copies(c + NBUF - 1, lax.rem(c + NBUF - 1, NBUF)):
                    cp.start()
            pl.when(c + NBUF - 1 < nvc)(prefetch)

            for cp in in_copies(c, b):
                cp.wait()

            def drain_prev_out():
                out_copy(c - NBUF, b).wait()
            pl.when(c >= NBUF)(drain_prev_out)

            r0 = c * R

            @plsc.parallel_loop(0, R, step=1, unroll=4)
            def row_body(j):
                valid = (r0 + j) < m
                for k in range(NV):
                    sl = pl.ds(k * 16, 16)
                    v = xb[b, j, sl] + p0b[b, j, sl] + p1b[b, j, sl]
                    ob[b, j, sl] = jnp.where(valid, jnp.maximum(v, 0.0), 0.0)

            out_copy(c, b).start()
            return 0

        lax.fori_loop(0, nvc, chunk_body, 0)

        for k in range(NBUF, 0, -1):
            def drain(k=k):
                out_copy(nvc - k, lax.rem(nvc - k, NBUF)).wait()
            pl.when(nvc >= k)(drain)

        ztail(lambda cp: cp.wait())
        return 0

    lax.fori_loop(0, GPW, graph_body, 0)


def kernel(atom_features, mol_slice, prev_activations):
    mesh = plsc.VectorSubcoreMesh(core_axis_name="c", subcore_axis_name="s")
    run = functools.partial(
        pl.kernel,
        mesh=mesh,
        out_type=jax.ShapeDtypeStruct((B, A, F), jnp.float32),
        scratch_types=[
            pltpu.VMEM((2 * GPW,), jnp.int32),
            pltpu.VMEM((NBUF, R, F), jnp.float32),
            pltpu.VMEM((NBUF, R, F), jnp.float32),
            pltpu.VMEM((NBUF, R, F), jnp.float32),
            pltpu.VMEM((NBUF, R, F), jnp.float32),
            pltpu.VMEM((ZR, F), jnp.float32),
            pltpu.SMEM((GPW,), jnp.int32),
            pltpu.SemaphoreType.DMA((NBUF,)),
            pltpu.SemaphoreType.DMA((NBUF,)),
            pltpu.SemaphoreType.DMA,
        ],
    )(_sc_body)
    return run(atom_features, mol_slice.reshape(-1), prev_activations)
